# baseline (speedup 1.0000x reference)
import functools

import jax
import jax.numpy as jnp
from jax import lax
from jax.experimental import pallas as pl
from jax.experimental.pallas import tpu as pltpu
from jax.experimental.pallas import tpu_sc as plsc

N = 10000
E = 320000
D = 128
NPAD = 10240
EPAD = 327680
NC = 2
NS = 16
NW = NC * NS
CHUNK = 128
KCH = EPAD // (NW * CHUNK)
HALF = KCH // 2
RPT = NPAD // NS

_mesh = plsc.VectorSubcoreMesh(core_axis_name="c", subcore_axis_name="s")


def _zero_vmem_2d(ref, rows, cols):
    zv = jnp.zeros((16,), jnp.float32)
    nv = cols // 16

    def body(i, carry):
        for j in range(nv):
            ref[i, pl.ds(j * 16, 16)] = zv
        return carry

    lax.fori_loop(0, rows, body, 0)


def _fill_rows(ref, rows, val):
    fv = jnp.full((16,), val, jnp.float32)

    def body(i, carry):
        ref[i] = fv
        return carry

    lax.fori_loop(0, rows, body, 0)


def _deg_body(dst_hbm, out_hbm, idx_v, obuf, deg_sh, sem):
    c = lax.axis_index("c")
    s = lax.axis_index("s")
    wid = c * NS + s

    _fill_rows(obuf, CHUNK, 0.0)
    for t in range(RPT // CHUNK):
        pltpu.sync_copy(obuf, deg_sh.at[pl.ds(s * RPT + t * CHUNK, CHUNK)])
    plsc.subcore_barrier()

    _fill_rows(obuf, CHUNK, 1.0)

    def body(j, carry):
        pltpu.sync_copy(obuf, deg_sh.at[idx_v.at[j]], add=True)
        return carry

    for half in range(2):
        pltpu.sync_copy(dst_hbm.at[wid, pl.ds(half * HALF, HALF)], idx_v)
        lax.fori_loop(0, HALF, body, 0)
    plsc.subcore_barrier()

    pltpu.sync_copy(deg_sh.at[pl.ds(s * RPT, RPT)],
                    out_hbm.at[c, pl.ds(s * RPT, RPT)])


_deg_kernel = functools.partial(
    pl.kernel,
    out_type=jax.ShapeDtypeStruct((NC, NPAD, 16), jnp.float32),
    mesh=_mesh,
    scratch_types=[
        pltpu.VMEM((HALF, CHUNK), jnp.int32),
        pltpu.VMEM((CHUNK, 16), jnp.float32),
        pltpu.VMEM_SHARED((NPAD, 16), jnp.float32),
        pltpu.SemaphoreType.DMA,
    ],
)(_deg_body)


def _hop_body(u_hbm, src_hbm, dst_hbm, out_hbm,
              idx_s, idx_d, buf0, buf1, acc_sh, sg0, sg1, ss0, ss1):
    c = lax.axis_index("c")
    s = lax.axis_index("s")
    wid = c * NS + s

    _zero_vmem_2d(buf0, CHUNK, D)
    for t in range(RPT // CHUNK):
        pltpu.sync_copy(buf0, acc_sh.at[pl.ds(s * RPT + t * CHUNK, CHUNK)])
    plsc.subcore_barrier()

    def body(j, carry):
        pltpu.make_async_copy(u_hbm.at[idx_s.at[j]], buf0, sg0).wait()
        cs0 = pltpu.async_copy(buf0, acc_sh.at[idx_d.at[j]], ss0, add=True)
        pltpu.make_async_copy(u_hbm.at[idx_s.at[j + 1]], buf1, sg1).wait()
        cs1 = pltpu.async_copy(buf1, acc_sh.at[idx_d.at[j + 1]], ss1,
                               add=True)
        cs0.wait()
        pltpu.async_copy(u_hbm.at[idx_s.at[j + 2]], buf0, sg0)
        cs1.wait()
        pltpu.async_copy(u_hbm.at[idx_s.at[j + 3]], buf1, sg1)
        return carry

    for half in range(2):
        pltpu.sync_copy(src_hbm.at[wid, pl.ds(half * HALF, HALF)], idx_s)
        pltpu.sync_copy(dst_hbm.at[wid, pl.ds(half * HALF, HALF)], idx_d)
        pltpu.async_copy(u_hbm.at[idx_s.at[0]], buf0, sg0)
        pltpu.async_copy(u_hbm.at[idx_s.at[1]], buf1, sg1)
        lax.fori_loop(0, (HALF - 2) // 2, lambda i, cr: body(i * 2, cr), 0)
        pltpu.make_async_copy(u_hbm.at[idx_s.at[HALF - 2]], buf0, sg0).wait()
        cs0 = pltpu.async_copy(buf0, acc_sh.at[idx_d.at[HALF - 2]], ss0,
                               add=True)
        pltpu.make_async_copy(u_hbm.at[idx_s.at[HALF - 1]], buf1, sg1).wait()
        cs1 = pltpu.async_copy(buf1, acc_sh.at[idx_d.at[HALF - 1]], ss1,
                               add=True)
        cs0.wait()
        cs1.wait()

    plsc.subcore_barrier()
    pltpu.sync_copy(acc_sh.at[pl.ds(s * RPT, RPT)],
                    out_hbm.at[c, pl.ds(s * RPT, RPT)])


_hop_kernel = functools.partial(
    pl.kernel,
    out_type=jax.ShapeDtypeStruct((NC, NPAD, D), jnp.float32),
    mesh=_mesh,
    scratch_types=[
        pltpu.VMEM((HALF, CHUNK), jnp.int32),
        pltpu.VMEM((HALF, CHUNK), jnp.int32),
        pltpu.VMEM((CHUNK, D), jnp.float32),
        pltpu.VMEM((CHUNK, D), jnp.float32),
        pltpu.VMEM_SHARED((NPAD, D), jnp.float32),
        pltpu.SemaphoreType.DMA,
        pltpu.SemaphoreType.DMA,
        pltpu.SemaphoreType.DMA,
        pltpu.SemaphoreType.DMA,
    ],
)(_hop_body)


_RB = 1280
_row_spec = pl.BlockSpec((_RB, D), lambda i: (i, 0))
_pair_spec = pl.BlockSpec((NC, _RB, D), lambda i: (0, i, 0))


def _scale1_body(deg_ref, x_ref, u_ref, r_ref):
    i = pl.program_id(0)
    deg = deg_ref[0] + deg_ref[1]
    deg_tot = deg[:, 0:1] + 1.0
    rows = lax.broadcasted_iota(jnp.int32, (_RB, 1), 0) + i * _RB
    dinv = jnp.where(rows < N, lax.rsqrt(deg_tot), 0.0)
    r = jnp.broadcast_to(dinv, (_RB, D))
    r_ref[...] = r
    u_ref[...] = r * x_ref[...]


def _tc_scale1(deg, x_pad):
    return pl.pallas_call(
        _scale1_body,
        grid=(NPAD // _RB,),
        in_specs=[pl.BlockSpec((NC, _RB, 16), lambda i: (0, i, 0)), _row_spec],
        out_specs=[_row_spec, _row_spec],
        out_shape=[jax.ShapeDtypeStruct((NPAD, D), jnp.float32),
                   jax.ShapeDtypeStruct((NPAD, D), jnp.float32)],
    )(deg, x_pad)


def _scale2_body(v_ref, u_ref, r_ref, w_ref):
    r = r_ref[...]
    w_ref[...] = r * r * (v_ref[0] + v_ref[1] + u_ref[...])


def _tc_scale2(v, u, r):
    return pl.pallas_call(
        _scale2_body,
        grid=(NPAD // _RB,),
        in_specs=[_pair_spec, _row_spec, _row_spec],
        out_specs=_row_spec,
        out_shape=jax.ShapeDtypeStruct((NPAD, D), jnp.float32),
    )(v, u, r)


def _final_body(z_ref, w_ref, r_ref, wmat_ref, b_ref, o_ref):
    h = r_ref[...] * (z_ref[0] + z_ref[1] + w_ref[...])
    o = jnp.dot(h, wmat_ref[...], preferred_element_type=jnp.float32)
    o = o + b_ref[...]
    m = jnp.max(o, axis=-1, keepdims=True)
    ex = jnp.exp(o - m)
    lse = jnp.log(jnp.sum(ex, axis=-1, keepdims=True)) + m
    o_ref[...] = o - lse


def _tc_final(z, w, r, wmat, b):
    return pl.pallas_call(
        _final_body,
        grid=(NPAD // _RB,),
        in_specs=[_pair_spec, _row_spec, _row_spec,
                  pl.BlockSpec((D, D), lambda i: (0, 0)),
                  pl.BlockSpec((1, D), lambda i: (0, 0))],
        out_specs=_row_spec,
        out_shape=jax.ShapeDtypeStruct((NPAD, D), jnp.float32),
    )(z, w, r, wmat, b)


def kernel(x, edge_index, W, b):
    x_pad = jnp.pad(x, ((0, NPAD - N), (0, 0)))
    pad = N + (jnp.arange(EPAD - E, dtype=edge_index.dtype) % (NPAD - N))
    src = jnp.concatenate([edge_index[0], pad]).reshape(NW, KCH, CHUNK)
    dst = jnp.concatenate([edge_index[1], pad]).reshape(NW, KCH, CHUNK)

    deg = _deg_kernel(dst)
    u, r = _tc_scale1(deg, x_pad)
    v = _hop_kernel(u, src, dst)
    w = _tc_scale2(v, u, r)
    z = _hop_kernel(w, src, dst)
    out = _tc_final(z, w, r, W, jnp.reshape(b, (1, D)))
    return out[:N]


# speedup vs baseline: 1.0908x; 1.0908x over previous
import functools

import jax
import jax.numpy as jnp
from jax import lax
from jax.experimental import pallas as pl
from jax.experimental.pallas import tpu as pltpu
from jax.experimental.pallas import tpu_sc as plsc

N = 10000
E = 320000
D = 128
NPAD = 10240
EPAD = 327680
NC = 2
NS = 16
NW = NC * NS
CHUNK = 128
KCH = EPAD // (NW * CHUNK)
HALF = KCH // 2
RPT = NPAD // NS

_mesh = plsc.VectorSubcoreMesh(core_axis_name="c", subcore_axis_name="s")


def _zero_vmem_2d(ref, rows, cols):
    zv = jnp.zeros((16,), jnp.float32)
    nv = cols // 16

    def body(i, carry):
        for j in range(nv):
            ref[i, pl.ds(j * 16, 16)] = zv
        return carry

    lax.fori_loop(0, rows, body, 0)


def _fill_rows(ref, rows, val):
    fv = jnp.full((16,), val, jnp.float32)

    def body(i, carry):
        ref[i] = fv
        return carry

    lax.fori_loop(0, rows, body, 0)


def _deg_body(dst_hbm, out_hbm, idx_v, obuf, deg_sh, sem):
    c = lax.axis_index("c")
    s = lax.axis_index("s")
    wid = c * NS + s

    _fill_rows(obuf, CHUNK, 0.0)
    for t in range(RPT // CHUNK):
        pltpu.sync_copy(obuf, deg_sh.at[pl.ds(s * RPT + t * CHUNK, CHUNK)])
    plsc.subcore_barrier()

    _fill_rows(obuf, CHUNK, 1.0)

    def body(j, carry):
        pltpu.sync_copy(obuf, deg_sh.at[idx_v.at[j]], add=True)
        return carry

    for half in range(2):
        pltpu.sync_copy(dst_hbm.at[wid, pl.ds(half * HALF, HALF)], idx_v)
        lax.fori_loop(0, HALF, body, 0)
    plsc.subcore_barrier()

    pltpu.sync_copy(deg_sh.at[pl.ds(s * RPT, RPT)],
                    out_hbm.at[c, pl.ds(s * RPT, RPT)])


_deg_kernel = functools.partial(
    pl.kernel,
    out_type=jax.ShapeDtypeStruct((NC, NPAD, 16), jnp.float32),
    mesh=_mesh,
    scratch_types=[
        pltpu.VMEM((HALF, CHUNK), jnp.int32),
        pltpu.VMEM((CHUNK, 16), jnp.float32),
        pltpu.VMEM_SHARED((NPAD, 16), jnp.float32),
        pltpu.SemaphoreType.DMA,
    ],
)(_deg_body)


def _hop_body(u_hbm, src_hbm, dst_hbm, out_hbm,
              idx_s, idx_d, buf0, buf1, acc_sh, sg0, sg1):
    c = lax.axis_index("c")
    s = lax.axis_index("s")
    wid = c * NS + s

    _zero_vmem_2d(buf0, CHUNK, D)
    for t in range(RPT // CHUNK):
        pltpu.sync_copy(buf0, acc_sh.at[pl.ds(s * RPT + t * CHUNK, CHUNK)])
    plsc.subcore_barrier()

    def body(j, carry):
        pltpu.async_copy(u_hbm.at[idx_s.at[j + 1]], buf1, sg1)
        pltpu.sync_copy(buf0, acc_sh.at[idx_d.at[j]], add=True)
        pltpu.make_async_copy(u_hbm.at[idx_s.at[j + 1]], buf1, sg1).wait()
        pltpu.async_copy(u_hbm.at[idx_s.at[j + 2]], buf0, sg0)
        pltpu.sync_copy(buf1, acc_sh.at[idx_d.at[j + 1]], add=True)
        pltpu.make_async_copy(u_hbm.at[idx_s.at[j + 2]], buf0, sg0).wait()
        return carry

    for half in range(2):
        pltpu.sync_copy(src_hbm.at[wid, pl.ds(half * HALF, HALF)], idx_s)
        pltpu.sync_copy(dst_hbm.at[wid, pl.ds(half * HALF, HALF)], idx_d)
        pltpu.sync_copy(u_hbm.at[idx_s.at[0]], buf0)
        lax.fori_loop(0, (HALF - 2) // 2, lambda i, cr: body(i * 2, cr), 0)
        pltpu.async_copy(u_hbm.at[idx_s.at[HALF - 1]], buf1, sg1)
        pltpu.sync_copy(buf0, acc_sh.at[idx_d.at[HALF - 2]], add=True)
        pltpu.make_async_copy(u_hbm.at[idx_s.at[HALF - 1]], buf1, sg1).wait()
        pltpu.sync_copy(buf1, acc_sh.at[idx_d.at[HALF - 1]], add=True)

    plsc.subcore_barrier()
    pltpu.sync_copy(acc_sh.at[pl.ds(s * RPT, RPT)],
                    out_hbm.at[c, pl.ds(s * RPT, RPT)])


_hop_kernel = functools.partial(
    pl.kernel,
    out_type=jax.ShapeDtypeStruct((NC, NPAD, D), jnp.float32),
    mesh=_mesh,
    scratch_types=[
        pltpu.VMEM((HALF, CHUNK), jnp.int32),
        pltpu.VMEM((HALF, CHUNK), jnp.int32),
        pltpu.VMEM((CHUNK, D), jnp.float32),
        pltpu.VMEM((CHUNK, D), jnp.float32),
        pltpu.VMEM_SHARED((NPAD, D), jnp.float32),
        pltpu.SemaphoreType.DMA,
        pltpu.SemaphoreType.DMA,
    ],
)(_hop_body)


_RB = 1280
_row_spec = pl.BlockSpec((_RB, D), lambda i: (i, 0))
_pair_spec = pl.BlockSpec((NC, _RB, D), lambda i: (0, i, 0))


_rcol_spec = pl.BlockSpec((_RB, 16), lambda i: (i, 0))


def _scale1_body(deg_ref, x_ref, u_ref, r_ref):
    i = pl.program_id(0)
    deg = deg_ref[0] + deg_ref[1]
    deg_tot = deg[:, 0:1] + 1.0
    rows = lax.broadcasted_iota(jnp.int32, (_RB, 1), 0) + i * _RB
    dinv = jnp.where(rows < N, lax.rsqrt(deg_tot), 0.0)
    r_ref[...] = jnp.broadcast_to(dinv, (_RB, 16))
    u_ref[...] = jnp.where(rows < N, dinv * x_ref[...], 0.0)


def _tc_scale1(deg, x):
    return pl.pallas_call(
        _scale1_body,
        grid=(NPAD // _RB,),
        in_specs=[pl.BlockSpec((NC, _RB, 16), lambda i: (0, i, 0)), _row_spec],
        out_specs=[_row_spec, _rcol_spec],
        out_shape=[jax.ShapeDtypeStruct((NPAD, D), jnp.float32),
                   jax.ShapeDtypeStruct((NPAD, 16), jnp.float32)],
    )(deg, x)


def _scale2_body(v_ref, u_ref, r_ref, w_ref):
    r = r_ref[:, 0:1]
    w_ref[...] = r * r * (v_ref[0] + v_ref[1] + u_ref[...])


def _tc_scale2(v, u, r):
    return pl.pallas_call(
        _scale2_body,
        grid=(NPAD // _RB,),
        in_specs=[_pair_spec, _row_spec, _rcol_spec],
        out_specs=_row_spec,
        out_shape=jax.ShapeDtypeStruct((NPAD, D), jnp.float32),
    )(v, u, r)


def _final_body(z_ref, w_ref, r_ref, wmat_ref, b_ref, o_ref):
    h = r_ref[:, 0:1] * (z_ref[0] + z_ref[1] + w_ref[...])
    o = jnp.dot(h, wmat_ref[...], preferred_element_type=jnp.float32)
    o = o + b_ref[...]
    m = jnp.max(o, axis=-1, keepdims=True)
    ex = jnp.exp(o - m)
    lse = jnp.log(jnp.sum(ex, axis=-1, keepdims=True)) + m
    o_ref[...] = o - lse


def _tc_final(z, w, r, wmat, b):
    return pl.pallas_call(
        _final_body,
        grid=(NPAD // _RB,),
        in_specs=[_pair_spec, _row_spec, _rcol_spec,
                  pl.BlockSpec((D, D), lambda i: (0, 0)),
                  pl.BlockSpec((1, D), lambda i: (0, 0))],
        out_specs=_row_spec,
        out_shape=jax.ShapeDtypeStruct((NPAD, D), jnp.float32),
    )(z, w, r, wmat, b)


def kernel(x, edge_index, W, b):
    pad = N + (jnp.arange(EPAD - E, dtype=edge_index.dtype) % (NPAD - N))
    src = jnp.concatenate([edge_index[0], pad]).reshape(NW, KCH, CHUNK)
    dst = jnp.concatenate([edge_index[1], pad]).reshape(NW, KCH, CHUNK)

    deg = _deg_kernel(dst)
    u, r = _tc_scale1(deg, x)
    v = _hop_kernel(u, src, dst)
    w = _tc_scale2(v, u, r)
    z = _hop_kernel(w, src, dst)
    out = _tc_final(z, w, r, W, jnp.reshape(b, (1, D)))
    return out[:N]


# speedup vs baseline: 1.1107x; 1.0183x over previous
import functools

import jax
import jax.numpy as jnp
from jax import lax
from jax.experimental import pallas as pl
from jax.experimental.pallas import tpu as pltpu
from jax.experimental.pallas import tpu_sc as plsc

N = 10000
E = 320000
D = 128
NPAD = 10240
EPAD = 327680
NC = 2
NS = 16
NW = NC * NS
CHUNK = 128
KCH = EPAD // (NW * CHUNK)
HALF = KCH // 2
RPT = NPAD // NS

_mesh = plsc.VectorSubcoreMesh(core_axis_name="c", subcore_axis_name="s")


def _zero_vmem_2d(ref, rows, cols):
    zv = jnp.zeros((16,), jnp.float32)
    nv = cols // 16

    def body(i, carry):
        for j in range(nv):
            ref[i, pl.ds(j * 16, 16)] = zv
        return carry

    lax.fori_loop(0, rows, body, 0)


def _fill_rows(ref, rows, val):
    fv = jnp.full((16,), val, jnp.float32)

    def body(i, carry):
        ref[i] = fv
        return carry

    lax.fori_loop(0, rows, body, 0)


def _deg_body(ed_hbm, out_hbm, idx_v, obuf, deg_sh, sem):
    c = lax.axis_index("c")
    s = lax.axis_index("s")
    wid = c * NS + s

    _fill_rows(obuf, CHUNK, 0.0)
    for t in range(RPT // CHUNK):
        pltpu.sync_copy(obuf, deg_sh.at[pl.ds(s * RPT + t * CHUNK, CHUNK)])
    plsc.subcore_barrier()

    _fill_rows(obuf, CHUNK, 1.0)

    def body(j, carry):
        pltpu.sync_copy(obuf, deg_sh.at[idx_v.at[j]], add=True)
        return carry

    for half in range(2):
        pltpu.sync_copy(ed_hbm.at[1, wid, pl.ds(half * HALF, HALF)], idx_v)
        lax.fori_loop(0, HALF, body, 0)
    plsc.subcore_barrier()

    pltpu.sync_copy(deg_sh.at[pl.ds(s * RPT, RPT)],
                    out_hbm.at[c, pl.ds(s * RPT, RPT)])


_deg_kernel = functools.partial(
    pl.kernel,
    out_type=jax.ShapeDtypeStruct((NC, NPAD, 16), jnp.float32),
    mesh=_mesh,
    scratch_types=[
        pltpu.VMEM((HALF, CHUNK), jnp.int32),
        pltpu.VMEM((CHUNK, 16), jnp.float32),
        pltpu.VMEM_SHARED((NPAD, 16), jnp.float32),
        pltpu.SemaphoreType.DMA,
    ],
)(_deg_body)


def _hop_body(u_hbm, ed_hbm, out_hbm,
              idx_s, idx_d, buf0, buf1, acc_sh, sg0, sg1):
    c = lax.axis_index("c")
    s = lax.axis_index("s")
    wid = c * NS + s

    _zero_vmem_2d(buf0, CHUNK, D)
    for t in range(RPT // CHUNK):
        pltpu.sync_copy(buf0, acc_sh.at[pl.ds(s * RPT + t * CHUNK, CHUNK)])
    plsc.subcore_barrier()

    def body(j, carry):
        pltpu.async_copy(u_hbm.at[idx_s.at[j + 1]], buf1, sg1)
        pltpu.sync_copy(buf0, acc_sh.at[idx_d.at[j]], add=True)
        pltpu.make_async_copy(u_hbm.at[idx_s.at[j + 1]], buf1, sg1).wait()
        pltpu.async_copy(u_hbm.at[idx_s.at[j + 2]], buf0, sg0)
        pltpu.sync_copy(buf1, acc_sh.at[idx_d.at[j + 1]], add=True)
        pltpu.make_async_copy(u_hbm.at[idx_s.at[j + 2]], buf0, sg0).wait()
        return carry

    for half in range(2):
        pltpu.sync_copy(ed_hbm.at[0, wid, pl.ds(half * HALF, HALF)], idx_s)
        pltpu.sync_copy(ed_hbm.at[1, wid, pl.ds(half * HALF, HALF)], idx_d)
        pltpu.sync_copy(u_hbm.at[idx_s.at[0]], buf0)
        lax.fori_loop(0, (HALF - 2) // 2, lambda i, cr: body(i * 2, cr), 0)
        pltpu.async_copy(u_hbm.at[idx_s.at[HALF - 1]], buf1, sg1)
        pltpu.sync_copy(buf0, acc_sh.at[idx_d.at[HALF - 2]], add=True)
        pltpu.make_async_copy(u_hbm.at[idx_s.at[HALF - 1]], buf1, sg1).wait()
        pltpu.sync_copy(buf1, acc_sh.at[idx_d.at[HALF - 1]], add=True)

    plsc.subcore_barrier()
    pltpu.sync_copy(acc_sh.at[pl.ds(s * RPT, RPT)],
                    out_hbm.at[c, pl.ds(s * RPT, RPT)])


_hop_kernel = functools.partial(
    pl.kernel,
    out_type=jax.ShapeDtypeStruct((NC, NPAD, D), jnp.float32),
    mesh=_mesh,
    scratch_types=[
        pltpu.VMEM((HALF, CHUNK), jnp.int32),
        pltpu.VMEM((HALF, CHUNK), jnp.int32),
        pltpu.VMEM((CHUNK, D), jnp.float32),
        pltpu.VMEM((CHUNK, D), jnp.float32),
        pltpu.VMEM_SHARED((NPAD, D), jnp.float32),
        pltpu.SemaphoreType.DMA,
        pltpu.SemaphoreType.DMA,
    ],
)(_hop_body)


_RB = 1280
_row_spec = pl.BlockSpec((_RB, D), lambda i: (i, 0))
_pair_spec = pl.BlockSpec((NC, _RB, D), lambda i: (0, i, 0))


_rcol_spec = pl.BlockSpec((_RB, 16), lambda i: (i, 0))


def _scale1_body(deg_ref, x_ref, u_ref, r_ref):
    i = pl.program_id(0)
    deg = deg_ref[0] + deg_ref[1]
    deg_tot = deg[:, 0:1] + 1.0
    rows = lax.broadcasted_iota(jnp.int32, (_RB, 1), 0) + i * _RB
    dinv = jnp.where(rows < N, lax.rsqrt(deg_tot), 0.0)
    r_ref[...] = jnp.broadcast_to(dinv, (_RB, 16))
    u_ref[...] = jnp.where(rows < N, dinv * x_ref[...], 0.0)


def _tc_scale1(deg, x):
    return pl.pallas_call(
        _scale1_body,
        grid=(NPAD // _RB,),
        in_specs=[pl.BlockSpec((NC, _RB, 16), lambda i: (0, i, 0)), _row_spec],
        out_specs=[_row_spec, _rcol_spec],
        out_shape=[jax.ShapeDtypeStruct((NPAD, D), jnp.float32),
                   jax.ShapeDtypeStruct((NPAD, 16), jnp.float32)],
    )(deg, x)


def _scale2_body(v_ref, u_ref, r_ref, w_ref):
    r = r_ref[:, 0:1]
    w_ref[...] = r * r * (v_ref[0] + v_ref[1] + u_ref[...])


def _tc_scale2(v, u, r):
    return pl.pallas_call(
        _scale2_body,
        grid=(NPAD // _RB,),
        in_specs=[_pair_spec, _row_spec, _rcol_spec],
        out_specs=_row_spec,
        out_shape=jax.ShapeDtypeStruct((NPAD, D), jnp.float32),
    )(v, u, r)


def _final_body(z_ref, w_ref, r_ref, wmat_ref, b_ref, o_ref):
    h = r_ref[:, 0:1] * (z_ref[0] + z_ref[1] + w_ref[...])
    o = jnp.dot(h, wmat_ref[...], preferred_element_type=jnp.float32)
    o = o + b_ref[...]
    m = jnp.max(o, axis=-1, keepdims=True)
    ex = jnp.exp(o - m)
    lse = jnp.log(jnp.sum(ex, axis=-1, keepdims=True)) + m
    o_ref[...] = o - lse


def _tc_final(z, w, r, wmat, b):
    return pl.pallas_call(
        _final_body,
        grid=(NPAD // _RB,),
        in_specs=[_pair_spec, _row_spec, _rcol_spec,
                  pl.BlockSpec((D, D), lambda i: (0, 0)),
                  pl.BlockSpec((1, D), lambda i: (0, 0))],
        out_specs=_row_spec,
        out_shape=jax.ShapeDtypeStruct((NPAD, D), jnp.float32),
    )(z, w, r, wmat, b)


def kernel(x, edge_index, W, b):
    pad = N + (jnp.arange(EPAD - E, dtype=edge_index.dtype) % (NPAD - N))
    ed = jnp.concatenate(
        [edge_index, jnp.broadcast_to(pad, (2, EPAD - E))], axis=1
    ).reshape(2, NW, KCH, CHUNK)

    deg = _deg_kernel(ed)
    u, r = _tc_scale1(deg, x)
    v = _hop_kernel(u, ed)
    w = _tc_scale2(v, u, r)
    z = _hop_kernel(w, ed)
    out = _tc_final(z, w, r, W, jnp.reshape(b, (1, D)))
    return out[:N]


# speedup vs baseline: 1.1205x; 1.0088x over previous
import functools

import jax
import jax.numpy as jnp
from jax import lax
from jax.experimental import pallas as pl
from jax.experimental.pallas import tpu as pltpu
from jax.experimental.pallas import tpu_sc as plsc

N = 10000
E = 320000
D = 128
NPAD = 10240
EPAD = 327680
NC = 2
NS = 16
NW = NC * NS
CHUNK = 128
KCH = EPAD // (NW * CHUNK)
HALF = KCH // 2
RPT = NPAD // NS

_mesh = plsc.VectorSubcoreMesh(core_axis_name="c", subcore_axis_name="s")


def _zero_vmem_2d(ref, rows, cols):
    zv = jnp.zeros((16,), jnp.float32)
    nv = cols // 16

    def body(i, carry):
        for j in range(nv):
            ref[i, pl.ds(j * 16, 16)] = zv
        return carry

    lax.fori_loop(0, rows, body, 0)


def _fill_rows(ref, rows, val):
    fv = jnp.full((16,), val, jnp.float32)

    def body(i, carry):
        ref[i] = fv
        return carry

    lax.fori_loop(0, rows, body, 0)


def _deg_body(ed_hbm, out_hbm, idx_v, obuf, deg_sh, sem):
    c = lax.axis_index("c")
    s = lax.axis_index("s")
    wid = c * NS + s

    _fill_rows(obuf, CHUNK, 0.0)
    for t in range(RPT // CHUNK):
        pltpu.sync_copy(obuf, deg_sh.at[pl.ds(s * RPT + t * CHUNK, CHUNK)])
    plsc.subcore_barrier()

    _fill_rows(obuf, CHUNK, 1.0)

    def body(j, carry):
        pltpu.sync_copy(obuf, deg_sh.at[idx_v.at[j]], add=True)
        return carry

    for half in range(2):
        pltpu.sync_copy(ed_hbm.at[1, wid, pl.ds(half * HALF, HALF)], idx_v)
        lax.fori_loop(0, HALF, body, 0)
    plsc.subcore_barrier()

    pltpu.sync_copy(deg_sh.at[pl.ds(s * RPT, RPT)],
                    out_hbm.at[c, pl.ds(s * RPT, RPT)])


_deg_kernel = functools.partial(
    pl.kernel,
    out_type=jax.ShapeDtypeStruct((NC, NPAD, 16), jnp.float32),
    mesh=_mesh,
    scratch_types=[
        pltpu.VMEM((HALF, CHUNK), jnp.int32),
        pltpu.VMEM((CHUNK, 16), jnp.float32),
        pltpu.VMEM_SHARED((NPAD, 16), jnp.float32),
        pltpu.SemaphoreType.DMA,
    ],
)(_deg_body)


def _hop_body(u_hbm, ed_hbm, out_hbm,
              idx_s, idx_d, buf0, buf1, acc_sh, sg0, sg1):
    c = lax.axis_index("c")
    s = lax.axis_index("s")
    wid = c * NS + s

    _zero_vmem_2d(buf0, CHUNK, D)
    for t in range(RPT // CHUNK):
        pltpu.sync_copy(buf0, acc_sh.at[pl.ds(s * RPT + t * CHUNK, CHUNK)])
    plsc.subcore_barrier()

    def body(j, carry):
        pltpu.async_copy(u_hbm.at[idx_s.at[j + 1]], buf1, sg1)
        pltpu.sync_copy(buf0, acc_sh.at[idx_d.at[j]], add=True)
        pltpu.make_async_copy(u_hbm.at[idx_s.at[j + 1]], buf1, sg1).wait()
        pltpu.async_copy(u_hbm.at[idx_s.at[j + 2]], buf0, sg0)
        pltpu.sync_copy(buf1, acc_sh.at[idx_d.at[j + 1]], add=True)
        pltpu.make_async_copy(u_hbm.at[idx_s.at[j + 2]], buf0, sg0).wait()
        return carry

    for half in range(2):
        pltpu.sync_copy(ed_hbm.at[0, wid, pl.ds(half * HALF, HALF)], idx_s)
        pltpu.sync_copy(ed_hbm.at[1, wid, pl.ds(half * HALF, HALF)], idx_d)
        pltpu.sync_copy(u_hbm.at[idx_s.at[0]], buf0)
        lax.fori_loop(0, (HALF - 2) // 2, lambda i, cr: body(i * 2, cr), 0)
        pltpu.async_copy(u_hbm.at[idx_s.at[HALF - 1]], buf1, sg1)
        pltpu.sync_copy(buf0, acc_sh.at[idx_d.at[HALF - 2]], add=True)
        pltpu.make_async_copy(u_hbm.at[idx_s.at[HALF - 1]], buf1, sg1).wait()
        pltpu.sync_copy(buf1, acc_sh.at[idx_d.at[HALF - 1]], add=True)

    plsc.subcore_barrier()
    pltpu.sync_copy(acc_sh.at[pl.ds(s * RPT, RPT)],
                    out_hbm.at[c, pl.ds(s * RPT, RPT)])


_hop_kernel = functools.partial(
    pl.kernel,
    out_type=jax.ShapeDtypeStruct((NC, NPAD, D), jnp.float32),
    mesh=_mesh,
    scratch_types=[
        pltpu.VMEM((HALF, CHUNK), jnp.int32),
        pltpu.VMEM((HALF, CHUNK), jnp.int32),
        pltpu.VMEM((CHUNK, D), jnp.float32),
        pltpu.VMEM((CHUNK, D), jnp.float32),
        pltpu.VMEM_SHARED((NPAD, D), jnp.float32),
        pltpu.SemaphoreType.DMA,
        pltpu.SemaphoreType.DMA,
    ],
)(_hop_body)


_RB = 1280
_row_spec = pl.BlockSpec((_RB, D), lambda i: (i, 0))
_pair_spec = pl.BlockSpec((NC, _RB, D), lambda i: (0, i, 0))


_rcol_spec = pl.BlockSpec((_RB, 16), lambda i: (i, 0))


def _scale1_body(deg_ref, x_ref, u_ref, r_ref):
    i = pl.program_id(0)
    deg = deg_ref[0] + deg_ref[1]
    deg_tot = deg[:, 0:1] + 1.0
    rows = lax.broadcasted_iota(jnp.int32, (_RB, 1), 0) + i * _RB
    dinv = jnp.where(rows < N, lax.rsqrt(deg_tot), 0.0)
    r_ref[...] = jnp.broadcast_to(dinv, (_RB, 16))
    u_ref[...] = jnp.where(rows < N, dinv * x_ref[...], 0.0)


def _tc_scale1(deg, x):
    return pl.pallas_call(
        _scale1_body,
        grid=(NPAD // _RB,),
        in_specs=[pl.BlockSpec((NC, _RB, 16), lambda i: (0, i, 0)), _row_spec],
        out_specs=[_row_spec, _rcol_spec],
        out_shape=[jax.ShapeDtypeStruct((NPAD, D), jnp.float32),
                   jax.ShapeDtypeStruct((NPAD, 16), jnp.float32)],
    )(deg, x)


def _scale2_body(v_ref, u_ref, r_ref, w_ref):
    r = r_ref[:, 0:1]
    w_ref[...] = r * r * (v_ref[0] + v_ref[1] + u_ref[...])


def _tc_scale2(v, u, r):
    return pl.pallas_call(
        _scale2_body,
        grid=(NPAD // _RB,),
        in_specs=[_pair_spec, _row_spec, _rcol_spec],
        out_specs=_row_spec,
        out_shape=jax.ShapeDtypeStruct((NPAD, D), jnp.float32),
    )(v, u, r)


def _final_body(z_ref, w_ref, r_ref, wmat_ref, b_ref, o_ref):
    h = r_ref[:, 0:1] * (z_ref[0] + z_ref[1] + w_ref[...])
    o = jnp.dot(h, wmat_ref[...], preferred_element_type=jnp.float32)
    o = o + b_ref[...]
    m = jnp.max(o, axis=-1, keepdims=True)
    ex = jnp.exp(o - m)
    lse = jnp.log(jnp.sum(ex, axis=-1, keepdims=True)) + m
    o_ref[...] = o - lse


_FB = 1000


def _tc_final(z, w, r, wmat, b):
    return pl.pallas_call(
        _final_body,
        grid=(N // _FB,),
        in_specs=[pl.BlockSpec((NC, _FB, D), lambda i: (0, i, 0)),
                  pl.BlockSpec((_FB, D), lambda i: (i, 0)),
                  pl.BlockSpec((_FB, 16), lambda i: (i, 0)),
                  pl.BlockSpec((D, D), lambda i: (0, 0)),
                  pl.BlockSpec((1, D), lambda i: (0, 0))],
        out_specs=pl.BlockSpec((_FB, D), lambda i: (i, 0)),
        out_shape=jax.ShapeDtypeStruct((N, D), jnp.float32),
    )(z, w, r, wmat, b)


def kernel(x, edge_index, W, b):
    pad = N + (jnp.arange(EPAD - E, dtype=edge_index.dtype) % (NPAD - N))
    ed = jnp.concatenate(
        [edge_index, jnp.broadcast_to(pad, (2, EPAD - E))], axis=1
    ).reshape(2, NW, KCH, CHUNK)

    deg = _deg_kernel(ed)
    u, r = _tc_scale1(deg, x)
    v = _hop_kernel(u, ed)
    w = _tc_scale2(v, u, r)
    z = _hop_kernel(w, ed)
    return _tc_final(z, w, r, W, jnp.reshape(b, (1, D)))
